# baseline (speedup 1.0000x reference)
import functools

import jax
import jax.numpy as jnp
from jax import lax
from jax.experimental import pallas as pl
from jax.experimental.pallas import tpu as pltpu
from jax.experimental.pallas import tpu_sc as plsc

N = 10000
E = 320000
R = 16
NB = 12
D = 128
NPAD = 10240
EPAD = 327680
CH = 512
IB = CH // 128
N_TILES = 32
E_TILE = EPAD // N_TILES
N_CHUNK = E_TILE // CH
E_SUB = EPAD // 16
C_CHUNK = E_SUB // CH

_mesh = plsc.VectorSubcoreMesh(core_axis_name="c", subcore_axis_name="s")


def _iota16():
    return lax.iota(jnp.int32, 16)


CH1 = 128
NC1 = E_SUB // CH1


@functools.partial(
    pl.kernel,
    out_type=jax.ShapeDtypeStruct((NPAD * 16, 128), jnp.float32),
    mesh=_mesh,
    scratch_types=[
        pltpu.VMEM_SHARED((NPAD, 128), jnp.float32),
        pltpu.VMEM((CH1, 128), jnp.float32),
        pltpu.VMEM((1, 128), jnp.int32),
        pltpu.VMEM((CH1,), jnp.int32),
        pltpu.VMEM((8, 128), jnp.float32),
    ],
)
def _p1_counts(dst2d, et, rcnt_out, cnt_sh, oh, dstb, tb, cb):
    c = lax.axis_index("c")
    s = lax.axis_index("s")
    wid = s * 2 + c
    zeros16 = jnp.zeros((16,), jnp.float32)
    iota = _iota16()

    def zrow(i, carry):
        for k in range(8):
            oh[i, pl.ds(k * 16, 16)] = zeros16
        return carry
    lax.fori_loop(0, CH1, zrow, 0)
    rows_per = NPAD // 16

    def zacc(i, carry):
        pltpu.sync_copy(oh, cnt_sh.at[pl.ds(s * rows_per + i * CH1, CH1)])
        return carry
    lax.fori_loop(0, rows_per // CH1, zacc, 0)
    plsc.subcore_barrier()

    ebase = s * E_SUB
    rbase = s * (E_SUB // 128)

    def chunk(ch, carry):
        pltpu.sync_copy(dst2d.at[pl.ds(rbase + ch, 1)], dstb)
        pltpu.sync_copy(et.at[pl.ds(ebase + ch * CH1, CH1)], tb)

        def put(j, carry):
            tvec = tb[pl.ds(j * 16, 16)]
            for i in range(16):
                ti = tvec[i]
                oh[j * 16 + i, pl.ds(0, 16)] = jnp.where(iota == ti, 1.0, 0.0)
            return carry
        lax.fori_loop(0, CH1 // 16, put, 0)
        pltpu.sync_copy(oh, cnt_sh.at[dstb.at[0]], add=True)
        return carry
    lax.fori_loop(0, NC1, chunk, 0)
    plsc.subcore_barrier()

    orow = NPAD // N_TILES

    def egrp(grp, carry):
        pltpu.sync_copy(cnt_sh.at[pl.ds(wid * orow + grp * 8, 8)], cb)
        def erow(rr, carry):
            cvec = cb[rr, pl.ds(0, 16)]
            rvec = 1.0 / cvec
            for i in range(16):
                ri = jnp.full((16,), rvec[i], jnp.float32)
                for k in range(8):
                    oh[rr * 16 + i, pl.ds(k * 16, 16)] = ri
            return carry
        lax.fori_loop(0, 8, erow, 0)
        pltpu.sync_copy(
            oh, rcnt_out.at[pl.ds((wid * orow + grp * 8) * 16, CH1)])
        return carry
    lax.fori_loop(0, orow // 8, egrp, 0)


@functools.partial(
    pl.kernel,
    out_type=(
        jax.ShapeDtypeStruct((EPAD,), jnp.int32),
        jax.ShapeDtypeStruct((EPAD,), jnp.float32),
    ),
    mesh=_mesh,
    scratch_types=[
        pltpu.VMEM((CH,), jnp.int32),
        pltpu.VMEM((CH,), jnp.int32),
        pltpu.VMEM((CH,), jnp.int32),
        pltpu.VMEM((CH,), jnp.int32),
        pltpu.VMEM((IB, 128), jnp.int32),
        pltpu.VMEM((CH, 128), jnp.float32),
        pltpu.VMEM((CH,), jnp.float32),
        pltpu.SemaphoreType.DMA,
    ],
)
def _p2_indices(src, et, dst, rcnt, g_out, w_out,
                srcb, tb, db, gb, kb2, wsp, wb, sem):
    c = lax.axis_index("c")
    s = lax.axis_index("s")
    wid = s * 2 + c
    iota = _iota16()
    ebase = wid * E_TILE

    def chunk(ch, carry):
        pltpu.sync_copy(src.at[pl.ds(ebase + ch * CH, CH)], srcb)
        pltpu.sync_copy(et.at[pl.ds(ebase + ch * CH, CH)], tb)
        pltpu.sync_copy(dst.at[pl.ds(ebase + ch * CH, CH)], db)

        def grp(j, carry):
            tvec = tb[pl.ds(j * 16, 16)]
            svec = srcb[pl.ds(j * 16, 16)]
            dvec = db[pl.ds(j * 16, 16)]
            gb[pl.ds(j * 16, 16)] = tvec * N + svec
            kb2[j // 8, pl.ds((j % 8) * 16, 16)] = dvec * 16 + tvec
            return carry
        lax.fori_loop(0, CH // 16, grp, 0)
        cps = [pltpu.async_copy(rcnt.at[kb2.at[j]],
                                wsp.at[pl.ds(j * 128, 128)], sem)
               for j in range(IB)]
        for cp in cps:
            cp.wait()

        def ext(j, carry):
            wv = jnp.zeros((16,), jnp.float32)
            for i in range(16):
                row = wsp[j * 16 + i, pl.ds(0, 16)]
                wv = jnp.where(iota == i, row[0], wv)
            wb[pl.ds(j * 16, 16)] = wv
            return carry
        lax.fori_loop(0, CH // 16, ext, 0)
        pltpu.sync_copy(gb, g_out.at[pl.ds(ebase + ch * CH, CH)])
        pltpu.sync_copy(wb, w_out.at[pl.ds(ebase + ch * CH, CH)])
        return carry
    lax.fori_loop(0, N_CHUNK, chunk, 0)


CHS = 128
NCS = E_TILE // CHS
TOTC = 2 * NCS
SPLIT0 = 0.3
N0 = 2 * round(TOTC * SPLIT0 / 2)
N1 = TOTC - N0


@functools.partial(
    pl.kernel,
    out_type=jax.ShapeDtypeStruct((2, NPAD, D), jnp.float32),
    mesh=_mesh,
    scratch_types=[
        pltpu.VMEM_SHARED((NPAD, D), jnp.float32),
        pltpu.VMEM((CHS, D), jnp.float32),
        pltpu.VMEM((CHS, D), jnp.float32),
        pltpu.VMEM((1, 128), jnp.int32),
        pltpu.VMEM((1, 128), jnp.int32),
        pltpu.VMEM((1, 128), jnp.int32),
        pltpu.VMEM((1, 128), jnp.int32),
        pltpu.VMEM((CHS,), jnp.float32),
        pltpu.VMEM((CHS,), jnp.float32),
        pltpu.SemaphoreType.DMA,
        pltpu.SemaphoreType.DMA,
        pltpu.SemaphoreType.DMA,
        pltpu.SemaphoreType.DMA,
    ],
)
def _s_scatter(h2d, g2d, dst2d, w, parts, acc_sh,
               msg0, msg1, gb0, gb1, dstb0, dstb1, wb0, wb1,
               semg0, semg1, sems0, sems1):
    c = lax.axis_index("c")
    s = lax.axis_index("s")
    wid = s * 2 + c
    zeros16 = jnp.zeros((16,), jnp.float32)
    zeros16i = jnp.zeros((16,), jnp.int32)
    msgs = (msg0, msg1)
    gbs = (gb0, gb1)
    dstbs = (dstb0, dstb1)
    wbs = (wb0, wb1)
    semgs = (semg0, semg1)
    semss = (sems0, sems1)

    def zrow(i, carry):
        for k in range(D // 16):
            msg0[i, pl.ds(k * 16, 16)] = zeros16
            msg1[i, pl.ds(k * 16, 16)] = zeros16
        return carry
    lax.fori_loop(0, CHS, zrow, 0)
    for k in range(8):
        dstb0[0, pl.ds(k * 16, 16)] = zeros16i
        dstb1[0, pl.ds(k * 16, 16)] = zeros16i
    rows_per = NPAD // 16

    def zacc(i, carry):
        pltpu.sync_copy(msg0, acc_sh.at[pl.ds(s * rows_per + i * CHS, CHS)])
        return carry
    lax.fori_loop(0, rows_per // CHS, zacc, 0)
    plsc.subcore_barrier()

    pltpu.async_copy(msg0, acc_sh.at[dstb0.at[0]], sems0, add=True)
    pltpu.async_copy(msg1, acc_sh.at[dstb1.at[0]], sems1, add=True)

    cbase = s * TOTC + c * N0
    npairs = jnp.where(c == 0, N0 // 2, N1 // 2)
    rbase = cbase
    ebase = cbase * CHS

    def pair(i2, carry):
        gcps = []
        for b in range(2):
            ch = i2 * 2 + b
            pltpu.make_async_copy(
                msgs[b], acc_sh.at[dstbs[b].at[0]], semss[b]).wait()
            pltpu.sync_copy(g2d.at[pl.ds(rbase + ch, 1)], gbs[b])
            pltpu.sync_copy(dst2d.at[pl.ds(rbase + ch, 1)], dstbs[b])
            pltpu.sync_copy(w.at[pl.ds(ebase + ch * CHS, CHS)], wbs[b])
            gcps.append(
                pltpu.async_copy(h2d.at[gbs[b].at[0]], msgs[b], semgs[b]))
        for b in range(2):
            gcps[b].wait()
            msg = msgs[b]
            wb = wbs[b]

            def scale(g, carry):
                wv16 = wb[pl.ds(g * 16, 16)]
                for i in range(16):
                    r = g * 16 + i
                    wvi = wv16[i]
                    for k in range(D // 16):
                        msg[r, pl.ds(k * 16, 16)] = (
                            msg[r, pl.ds(k * 16, 16)] * wvi)
                return carry
            lax.fori_loop(0, CHS // 16, scale, 0)
            pltpu.async_copy(msg, acc_sh.at[dstbs[b].at[0]], semss[b],
                             add=True)
        return carry
    lax.fori_loop(0, npairs, pair, 0)
    for b in range(2):
        pltpu.make_async_copy(
            msgs[b], acc_sh.at[dstbs[b].at[0]], semss[b]).wait()
    plsc.subcore_barrier()

    pltpu.sync_copy(acc_sh.at[pl.ds(s * rows_per, rows_per)],
                    parts.at[c, pl.ds(s * rows_per, rows_per)])


def _wc_body(comp_ref, basis_ref, out_ref):
    r = pl.program_id(0)
    acc = jnp.zeros((D, D), jnp.float32)
    for b in range(NB):
        acc = acc + comp_ref[r, b] * basis_ref[b]
    out_ref[0] = acc


def _wc(comp, basis):
    return pl.pallas_call(
        _wc_body,
        grid=(R,),
        in_specs=[
            pl.BlockSpec(memory_space=pltpu.SMEM),
            pl.BlockSpec((NB, D, D), lambda r: (0, 0, 0)),
        ],
        out_specs=pl.BlockSpec((1, D, D), lambda r: (r, 0, 0)),
        out_shape=jax.ShapeDtypeStruct((R, D, D), jnp.float32),
    )(comp, basis)


_BN = 1000
_NBLK = N // _BN


def _mm_first_body(x_ref, wc_ref, root_ref, bias_ref, h_ref, init_ref):
    x = x_ref[...]
    init_ref[...] = (
        jnp.dot(x, root_ref[...], preferred_element_type=jnp.float32)
        + bias_ref[...]
    )
    for r in range(R):
        h_ref[r] = jnp.dot(x, wc_ref[r], preferred_element_type=jnp.float32)


def _mm_first(x, wc, root, bias):
    return pl.pallas_call(
        _mm_first_body,
        grid=(_NBLK,),
        in_specs=[
            pl.BlockSpec((_BN, D), lambda i: (i, 0)),
            pl.BlockSpec((R, D, D), lambda i: (0, 0, 0)),
            pl.BlockSpec((D, D), lambda i: (0, 0)),
            pl.BlockSpec((1, D), lambda i: (0, 0)),
        ],
        out_specs=[
            pl.BlockSpec((R, _BN, D), lambda i: (0, i, 0)),
            pl.BlockSpec((_BN, D), lambda i: (i, 0)),
        ],
        out_shape=[
            jax.ShapeDtypeStruct((R, N, D), jnp.float32),
            jax.ShapeDtypeStruct((N, D), jnp.float32),
        ],
    )(x, wc, root, bias)


def _mm_next_body(initp_ref, p0_ref, p1_ref, wc_ref, root_ref, bias_ref,
                  h_ref, init_ref, x_ref):
    x = jnp.maximum(initp_ref[...] + p0_ref[...] + p1_ref[...], 0.0)
    x_ref[...] = x
    init_ref[...] = (
        jnp.dot(x, root_ref[...], preferred_element_type=jnp.float32)
        + bias_ref[...]
    )
    for r in range(R):
        h_ref[r] = jnp.dot(x, wc_ref[r], preferred_element_type=jnp.float32)


def _mm_next(initp, p0, p1, wc, root, bias):
    blk = pl.BlockSpec((_BN, D), lambda i: (i, 0))
    return pl.pallas_call(
        _mm_next_body,
        grid=(_NBLK,),
        in_specs=[
            blk, blk, blk,
            pl.BlockSpec((R, D, D), lambda i: (0, 0, 0)),
            pl.BlockSpec((D, D), lambda i: (0, 0)),
            pl.BlockSpec((1, D), lambda i: (0, 0)),
        ],
        out_specs=[
            pl.BlockSpec((R, _BN, D), lambda i: (0, i, 0)),
            blk, blk,
        ],
        out_shape=[
            jax.ShapeDtypeStruct((R, N, D), jnp.float32),
            jax.ShapeDtypeStruct((N, D), jnp.float32),
            jax.ShapeDtypeStruct((N, D), jnp.float32),
        ],
    )(initp, p0, p1, wc, root, bias)


def _relu_body(initp_ref, p0_ref, p1_ref, x_ref):
    x_ref[...] = jnp.maximum(initp_ref[...] + p0_ref[...] + p1_ref[...], 0.0)


def _relu_sum(initp, p0, p1):
    blk = pl.BlockSpec((_BN, D), lambda i: (i, 0))
    return pl.pallas_call(
        _relu_body,
        grid=(_NBLK,),
        in_specs=[blk, blk, blk],
        out_specs=blk,
        out_shape=jax.ShapeDtypeStruct((N, D), jnp.float32),
    )(initp, p0, p1)


def kernel(adj_t, edge_types, emb, basis1, comp1, root1, bias1,
           basis2, comp2, root2, bias2, basis3, comp3, root3, bias3):
    src = adj_t[0].astype(jnp.int32)
    dst = adj_t[1].astype(jnp.int32)
    et = edge_types.astype(jnp.int32)
    pad = EPAD - E
    srcp = jnp.concatenate([src, jnp.zeros((pad,), jnp.int32)])
    dstp = jnp.concatenate([dst, jnp.full((pad,), NPAD - 1, jnp.int32)])
    etp = jnp.concatenate([et, jnp.zeros((pad,), jnp.int32)])
    dst2d = dstp.reshape(-1, 128)

    rcnt = _p1_counts(dst2d, etp)
    g, w = _p2_indices(srcp, etp, dstp, rcnt)
    g2d = g.reshape(-1, 128)

    def layer(x_or_initp, parts_prev, basis, comp, root, bias, first):
        wc = _wc(comp, basis)
        b2 = bias.reshape(1, D)
        if first:
            h, init = _mm_first(x_or_initp, wc, root, b2)
            xprev = None
        else:
            h, init, xprev = _mm_next(
                x_or_initp, parts_prev[0, :N], parts_prev[1, :N],
                wc, root, b2)
        parts = _s_scatter(h.reshape(R * N, D), g2d, dst2d, w)
        return init, parts, xprev

    init1, parts1, _ = layer(emb, None, basis1, comp1, root1, bias1, True)
    init2, parts2, x1 = layer(init1, parts1, basis2, comp2, root2, bias2, False)
    init3, parts3, x2 = layer(init2, parts2, basis3, comp3, root3, bias3, False)
    x3 = _relu_sum(init3, parts3[0, :N], parts3[1, :N])
    return jnp.concatenate([x3, x2, x1, emb], axis=1)


# speedup vs baseline: 1.2337x; 1.2337x over previous
import functools

import jax
import jax.numpy as jnp
from jax import lax
from jax.experimental import pallas as pl
from jax.experimental.pallas import tpu as pltpu
from jax.experimental.pallas import tpu_sc as plsc

N = 10000
E = 320000
R = 16
NB = 12
D = 128
NPAD = 10240
EPAD = 327680
CH = 512
IB = CH // 128
N_TILES = 32
E_TILE = EPAD // N_TILES
N_CHUNK = E_TILE // CH
E_SUB = EPAD // 16
C_CHUNK = E_SUB // CH

_mesh = plsc.VectorSubcoreMesh(core_axis_name="c", subcore_axis_name="s")


def _iota16():
    return lax.iota(jnp.int32, 16)


CH1 = 128
NC1 = E_SUB // CH1


@functools.partial(
    pl.kernel,
    out_type=jax.ShapeDtypeStruct((NPAD * 16, 128), jnp.float32),
    mesh=_mesh,
    scratch_types=[
        pltpu.VMEM_SHARED((NPAD, 128), jnp.float32),
        pltpu.VMEM((CH1, 128), jnp.float32),
        pltpu.VMEM((1, 128), jnp.int32),
        pltpu.VMEM((CH1,), jnp.int32),
        pltpu.VMEM((8, 128), jnp.float32),
    ],
)
def _p1_counts(dst2d, et, rcnt_out, cnt_sh, oh, dstb, tb, cb):
    c = lax.axis_index("c")
    s = lax.axis_index("s")
    wid = s * 2 + c
    zeros16 = jnp.zeros((16,), jnp.float32)
    iota = _iota16()

    def zrow(i, carry):
        for k in range(8):
            oh[i, pl.ds(k * 16, 16)] = zeros16
        return carry
    lax.fori_loop(0, CH1, zrow, 0)
    rows_per = NPAD // 16

    def zacc(i, carry):
        pltpu.sync_copy(oh, cnt_sh.at[pl.ds(s * rows_per + i * CH1, CH1)])
        return carry
    lax.fori_loop(0, rows_per // CH1, zacc, 0)
    plsc.subcore_barrier()

    ebase = s * E_SUB
    rbase = s * (E_SUB // 128)

    def chunk(ch, carry):
        pltpu.sync_copy(dst2d.at[pl.ds(rbase + ch, 1)], dstb)
        pltpu.sync_copy(et.at[pl.ds(ebase + ch * CH1, CH1)], tb)

        def put(j, carry):
            tvec = tb[pl.ds(j * 16, 16)]
            for i in range(16):
                ti = tvec[i]
                oh[j * 16 + i, pl.ds(0, 16)] = jnp.where(iota == ti, 1.0, 0.0)
            return carry
        lax.fori_loop(0, CH1 // 16, put, 0)
        pltpu.sync_copy(oh, cnt_sh.at[dstb.at[0]], add=True)
        return carry
    lax.fori_loop(0, NC1, chunk, 0)
    plsc.subcore_barrier()

    orow = NPAD // N_TILES

    def egrp(grp, carry):
        pltpu.sync_copy(cnt_sh.at[pl.ds(wid * orow + grp * 8, 8)], cb)
        def erow(rr, carry):
            cvec = cb[rr, pl.ds(0, 16)]
            rvec = 1.0 / cvec
            for i in range(16):
                ri = jnp.full((16,), rvec[i], jnp.float32)
                for k in range(8):
                    oh[rr * 16 + i, pl.ds(k * 16, 16)] = ri
            return carry
        lax.fori_loop(0, 8, erow, 0)
        pltpu.sync_copy(
            oh, rcnt_out.at[pl.ds((wid * orow + grp * 8) * 16, CH1)])
        return carry
    lax.fori_loop(0, orow // 8, egrp, 0)


@functools.partial(
    pl.kernel,
    out_type=(
        jax.ShapeDtypeStruct((EPAD,), jnp.int32),
        jax.ShapeDtypeStruct((EPAD,), jnp.float32),
    ),
    mesh=_mesh,
    scratch_types=[
        pltpu.VMEM((CH,), jnp.int32),
        pltpu.VMEM((CH,), jnp.int32),
        pltpu.VMEM((CH,), jnp.int32),
        pltpu.VMEM((CH,), jnp.int32),
        pltpu.VMEM((IB, 128), jnp.int32),
        pltpu.VMEM((CH, 128), jnp.float32),
        pltpu.VMEM((CH,), jnp.float32),
        pltpu.SemaphoreType.DMA,
    ],
)
def _p2_indices(src, et, dst, rcnt, g_out, w_out,
                srcb, tb, db, gb, kb2, wsp, wb, sem):
    c = lax.axis_index("c")
    s = lax.axis_index("s")
    wid = s * 2 + c
    iota = _iota16()
    ebase = wid * E_TILE

    def chunk(ch, carry):
        pltpu.sync_copy(src.at[pl.ds(ebase + ch * CH, CH)], srcb)
        pltpu.sync_copy(et.at[pl.ds(ebase + ch * CH, CH)], tb)
        pltpu.sync_copy(dst.at[pl.ds(ebase + ch * CH, CH)], db)

        def grp(j, carry):
            tvec = tb[pl.ds(j * 16, 16)]
            svec = srcb[pl.ds(j * 16, 16)]
            dvec = db[pl.ds(j * 16, 16)]
            gb[pl.ds(j * 16, 16)] = tvec * N + svec
            kb2[j // 8, pl.ds((j % 8) * 16, 16)] = dvec * 16 + tvec
            return carry
        lax.fori_loop(0, CH // 16, grp, 0)
        cps = [pltpu.async_copy(rcnt.at[kb2.at[j]],
                                wsp.at[pl.ds(j * 128, 128)], sem)
               for j in range(IB)]
        for cp in cps:
            cp.wait()

        def ext(j, carry):
            wv = jnp.zeros((16,), jnp.float32)
            for i in range(16):
                row = wsp[j * 16 + i, pl.ds(0, 16)]
                wv = jnp.where(iota == i, row[0], wv)
            wb[pl.ds(j * 16, 16)] = wv
            return carry
        lax.fori_loop(0, CH // 16, ext, 0)
        pltpu.sync_copy(gb, g_out.at[pl.ds(ebase + ch * CH, CH)])
        pltpu.sync_copy(wb, w_out.at[pl.ds(ebase + ch * CH, CH)])
        return carry
    lax.fori_loop(0, N_CHUNK, chunk, 0)


CHS = 128
NCS = E_TILE // CHS
TOTC = 2 * NCS
SPLIT0 = 0.7
N0 = 2 * round(TOTC * SPLIT0 / 2)
N1 = TOTC - N0


@functools.partial(
    pl.kernel,
    out_type=jax.ShapeDtypeStruct((2, NPAD, D), jnp.float32),
    mesh=_mesh,
    scratch_types=[
        pltpu.VMEM_SHARED((NPAD, D), jnp.float32),
        pltpu.VMEM((CHS, D), jnp.float32),
        pltpu.VMEM((CHS, D), jnp.float32),
        pltpu.VMEM((1, 128), jnp.int32),
        pltpu.VMEM((1, 128), jnp.int32),
        pltpu.VMEM((1, 128), jnp.int32),
        pltpu.VMEM((1, 128), jnp.int32),
        pltpu.VMEM((CHS,), jnp.float32),
        pltpu.VMEM((CHS,), jnp.float32),
        pltpu.SemaphoreType.DMA,
        pltpu.SemaphoreType.DMA,
        pltpu.SemaphoreType.DMA,
        pltpu.SemaphoreType.DMA,
    ],
)
def _s_scatter(h2d, g2d, dst2d, w, parts, acc_sh,
               msg0, msg1, gb0, gb1, dstb0, dstb1, wb0, wb1,
               semg0, semg1, sems0, sems1):
    c = lax.axis_index("c")
    s = lax.axis_index("s")
    wid = s * 2 + c
    zeros16 = jnp.zeros((16,), jnp.float32)
    zeros16i = jnp.zeros((16,), jnp.int32)
    msgs = (msg0, msg1)
    gbs = (gb0, gb1)
    dstbs = (dstb0, dstb1)
    wbs = (wb0, wb1)
    semgs = (semg0, semg1)
    semss = (sems0, sems1)

    def zrow(i, carry):
        for k in range(D // 16):
            msg0[i, pl.ds(k * 16, 16)] = zeros16
            msg1[i, pl.ds(k * 16, 16)] = zeros16
        return carry
    lax.fori_loop(0, CHS, zrow, 0)
    for k in range(8):
        dstb0[0, pl.ds(k * 16, 16)] = zeros16i
        dstb1[0, pl.ds(k * 16, 16)] = zeros16i
    rows_per = NPAD // 16

    def zacc(i, carry):
        pltpu.sync_copy(msg0, acc_sh.at[pl.ds(s * rows_per + i * CHS, CHS)])
        return carry
    lax.fori_loop(0, rows_per // CHS, zacc, 0)
    plsc.subcore_barrier()

    pltpu.async_copy(msg0, acc_sh.at[dstb0.at[0]], sems0, add=True)
    pltpu.async_copy(msg1, acc_sh.at[dstb1.at[0]], sems1, add=True)

    cbase = s * TOTC + c * N0
    npairs = jnp.where(c == 0, N0 // 2, N1 // 2)
    rbase = cbase
    ebase = cbase * CHS

    def pair(i2, carry):
        gcps = []
        for b in range(2):
            ch = i2 * 2 + b
            pltpu.make_async_copy(
                msgs[b], acc_sh.at[dstbs[b].at[0]], semss[b]).wait()
            pltpu.sync_copy(g2d.at[pl.ds(rbase + ch, 1)], gbs[b])
            pltpu.sync_copy(dst2d.at[pl.ds(rbase + ch, 1)], dstbs[b])
            pltpu.sync_copy(w.at[pl.ds(ebase + ch * CHS, CHS)], wbs[b])
            gcps.append(
                pltpu.async_copy(h2d.at[gbs[b].at[0]], msgs[b], semgs[b]))
        for b in range(2):
            gcps[b].wait()
            msg = msgs[b]
            wb = wbs[b]

            def scale(g, carry):
                wv16 = wb[pl.ds(g * 16, 16)]
                for i in range(16):
                    r = g * 16 + i
                    wvi = wv16[i]
                    for k in range(D // 16):
                        msg[r, pl.ds(k * 16, 16)] = (
                            msg[r, pl.ds(k * 16, 16)] * wvi)
                return carry
            lax.fori_loop(0, CHS // 16, scale, 0)
            pltpu.async_copy(msg, acc_sh.at[dstbs[b].at[0]], semss[b],
                             add=True)
        return carry
    lax.fori_loop(0, npairs, pair, 0)
    for b in range(2):
        pltpu.make_async_copy(
            msgs[b], acc_sh.at[dstbs[b].at[0]], semss[b]).wait()
    plsc.subcore_barrier()

    pltpu.sync_copy(acc_sh.at[pl.ds(s * rows_per, rows_per)],
                    parts.at[c, pl.ds(s * rows_per, rows_per)])


def _wc_body(comp_ref, basis_ref, out_ref):
    r = pl.program_id(0)
    acc = jnp.zeros((D, D), jnp.float32)
    for b in range(NB):
        acc = acc + comp_ref[r, b] * basis_ref[b]
    out_ref[0] = acc


def _wc(comp, basis):
    return pl.pallas_call(
        _wc_body,
        grid=(R,),
        in_specs=[
            pl.BlockSpec(memory_space=pltpu.SMEM),
            pl.BlockSpec((NB, D, D), lambda r: (0, 0, 0)),
        ],
        out_specs=pl.BlockSpec((1, D, D), lambda r: (r, 0, 0)),
        out_shape=jax.ShapeDtypeStruct((R, D, D), jnp.float32),
    )(comp, basis)


_BN = 1000
_NBLK = N // _BN


def _mm_first_body(x_ref, wc_ref, root_ref, bias_ref, h_ref, init_ref):
    x = x_ref[...]
    init_ref[...] = (
        jnp.dot(x, root_ref[...], preferred_element_type=jnp.float32)
        + bias_ref[...]
    )
    for r in range(R):
        h_ref[r] = jnp.dot(x, wc_ref[r], preferred_element_type=jnp.float32)


def _mm_first(x, wc, root, bias):
    return pl.pallas_call(
        _mm_first_body,
        grid=(_NBLK,),
        in_specs=[
            pl.BlockSpec((_BN, D), lambda i: (i, 0)),
            pl.BlockSpec((R, D, D), lambda i: (0, 0, 0)),
            pl.BlockSpec((D, D), lambda i: (0, 0)),
            pl.BlockSpec((1, D), lambda i: (0, 0)),
        ],
        out_specs=[
            pl.BlockSpec((R, _BN, D), lambda i: (0, i, 0)),
            pl.BlockSpec((_BN, D), lambda i: (i, 0)),
        ],
        out_shape=[
            jax.ShapeDtypeStruct((R, N, D), jnp.float32),
            jax.ShapeDtypeStruct((N, D), jnp.float32),
        ],
    )(x, wc, root, bias)


def _mm_next_body(initp_ref, p0_ref, p1_ref, wc_ref, root_ref, bias_ref,
                  h_ref, init_ref, x_ref):
    x = jnp.maximum(initp_ref[...] + p0_ref[...] + p1_ref[...], 0.0)
    x_ref[...] = x
    init_ref[...] = (
        jnp.dot(x, root_ref[...], preferred_element_type=jnp.float32)
        + bias_ref[...]
    )
    for r in range(R):
        h_ref[r] = jnp.dot(x, wc_ref[r], preferred_element_type=jnp.float32)


def _mm_next(initp, p0, p1, wc, root, bias):
    blk = pl.BlockSpec((_BN, D), lambda i: (i, 0))
    return pl.pallas_call(
        _mm_next_body,
        grid=(_NBLK,),
        in_specs=[
            blk, blk, blk,
            pl.BlockSpec((R, D, D), lambda i: (0, 0, 0)),
            pl.BlockSpec((D, D), lambda i: (0, 0)),
            pl.BlockSpec((1, D), lambda i: (0, 0)),
        ],
        out_specs=[
            pl.BlockSpec((R, _BN, D), lambda i: (0, i, 0)),
            blk, blk,
        ],
        out_shape=[
            jax.ShapeDtypeStruct((R, N, D), jnp.float32),
            jax.ShapeDtypeStruct((N, D), jnp.float32),
            jax.ShapeDtypeStruct((N, D), jnp.float32),
        ],
    )(initp, p0, p1, wc, root, bias)


def _relu_body(initp_ref, p0_ref, p1_ref, x_ref):
    x_ref[...] = jnp.maximum(initp_ref[...] + p0_ref[...] + p1_ref[...], 0.0)


def _relu_sum(initp, p0, p1):
    blk = pl.BlockSpec((_BN, D), lambda i: (i, 0))
    return pl.pallas_call(
        _relu_body,
        grid=(_NBLK,),
        in_specs=[blk, blk, blk],
        out_specs=blk,
        out_shape=jax.ShapeDtypeStruct((N, D), jnp.float32),
    )(initp, p0, p1)


def kernel(adj_t, edge_types, emb, basis1, comp1, root1, bias1,
           basis2, comp2, root2, bias2, basis3, comp3, root3, bias3):
    src = adj_t[0].astype(jnp.int32)
    dst = adj_t[1].astype(jnp.int32)
    et = edge_types.astype(jnp.int32)
    pad = EPAD - E
    srcp = jnp.concatenate([src, jnp.zeros((pad,), jnp.int32)])
    dstp = jnp.concatenate([dst, jnp.full((pad,), NPAD - 1, jnp.int32)])
    etp = jnp.concatenate([et, jnp.zeros((pad,), jnp.int32)])
    dst2d = dstp.reshape(-1, 128)

    rcnt = _p1_counts(dst2d, etp)
    g, w = _p2_indices(srcp, etp, dstp, rcnt)
    g2d = g.reshape(-1, 128)

    def layer(x_or_initp, parts_prev, basis, comp, root, bias, first):
        wc = _wc(comp, basis)
        b2 = bias.reshape(1, D)
        if first:
            h, init = _mm_first(x_or_initp, wc, root, b2)
            xprev = None
        else:
            h, init, xprev = _mm_next(
                x_or_initp, parts_prev[0, :N], parts_prev[1, :N],
                wc, root, b2)
        parts = _s_scatter(h.reshape(R * N, D), g2d, dst2d, w)
        return init, parts, xprev

    init1, parts1, _ = layer(emb, None, basis1, comp1, root1, bias1, True)
    init2, parts2, x1 = layer(init1, parts1, basis2, comp2, root2, bias2, False)
    init3, parts3, x2 = layer(init2, parts2, basis3, comp3, root3, bias3, False)
    x3 = _relu_sum(init3, parts3[0, :N], parts3[1, :N])
    return jnp.concatenate([x3, x2, x1, emb], axis=1)
